# baseline (speedup 1.0000x reference)
import functools

import jax
import jax.numpy as jnp
from jax import lax
from jax.experimental import pallas as pl
from jax.experimental.pallas import tpu as pltpu
from jax.experimental.pallas import tpu_sc as plsc

D_MODEL = 64
NUM_CORES = 2
NUM_SUBCORES = 16
NUM_WORKERS = NUM_CORES * NUM_SUBCORES


def _embed_call(n_rows, chunk, nbuf):
    assert n_rows % NUM_WORKERS == 0
    b_per_w = n_rows // NUM_WORKERS
    assert b_per_w % chunk == 0
    n_chunks = b_per_w // chunk
    assert n_chunks >= nbuf
    width = 2 * D_MODEL

    mesh = plsc.VectorSubcoreMesh(core_axis_name="c", subcore_axis_name="s")

    @functools.partial(
        pl.kernel,
        mesh=mesh,
        out_type=jax.ShapeDtypeStruct((n_rows, width), jnp.float32),
        scratch_types=[
            pltpu.VMEM((b_per_w,), jnp.int32),
            pltpu.VMEM((nbuf, chunk, width), jnp.float32),
            pltpu.SemaphoreType.DMA((nbuf,)),
            pltpu.SemaphoreType.DMA((nbuf,)),
        ],
    )
    def k(idx_hbm, table_hbm, out_hbm, idx_v, rows_v, gsem, ssem):
        wid = lax.axis_index("s") * NUM_CORES + lax.axis_index("c")
        base = wid * b_per_w
        pltpu.sync_copy(idx_hbm.at[pl.ds(base, b_per_w)], idx_v)

        def gather(c):
            b = c % nbuf
            return pltpu.make_async_copy(
                table_hbm.at[idx_v.at[pl.ds(c * chunk, chunk)]],
                rows_v.at[b],
                gsem.at[b],
            )

        def store(c):
            b = c % nbuf
            return pltpu.make_async_copy(
                rows_v.at[b],
                out_hbm.at[pl.ds(base + c * chunk, chunk)],
                ssem.at[b],
            )

        for c in range(nbuf - 1):
            gather(c).start()
        for c in range(n_chunks):
            nxt = c + nbuf - 1
            if nxt < n_chunks:
                if nxt >= nbuf:
                    store(nxt - nbuf).wait()
                gather(nxt).start()
            gather(c).wait()
            store(c).start()
        for c in range(n_chunks - nbuf, n_chunks):
            store(c).wait()

    return k


def _transpose_pad(table_t):
    d, v = table_t.shape
    blk = 2048
    grid = (v + blk - 1) // blk

    eye = jnp.eye(d, dtype=jnp.float32)

    def body(t_ref, eye_ref, w_ref):
        t = jax.lax.dot_general(
            t_ref[...], eye_ref[...],
            dimension_numbers=(((0,), (0,)), ((), ())),
            preferred_element_type=jnp.float32,
        )
        w_ref[...] = jnp.concatenate([t, jnp.zeros_like(t)], axis=1)

    return pl.pallas_call(
        body,
        grid=(grid,),
        in_specs=[
            pl.BlockSpec((d, blk), lambda j: (0, j)),
            pl.BlockSpec((d, d), lambda j: (0, 0)),
        ],
        out_specs=pl.BlockSpec((blk, 2 * d), lambda j: (j, 0)),
        out_shape=jax.ShapeDtypeStruct((v, 2 * d), jnp.float32),
        compiler_params=pltpu.CompilerParams(
            dimension_semantics=("arbitrary",),
        ),
    )(table_t, eye)


def kernel(x, table):
    b, s = x.shape
    idx = x.T.reshape(-1).astype(jnp.int32)
    table2 = _transpose_pad(table.T)
    wide = _embed_call(idx.shape[0], 200, 4)(idx, table2)
    out = wide[:, :D_MODEL]
    return out.reshape(s, b, D_MODEL).transpose(1, 0, 2)


# speedup vs baseline: 1.3161x; 1.3161x over previous
import functools

import jax
import jax.numpy as jnp
from jax import lax
from jax.experimental import pallas as pl
from jax.experimental.pallas import tpu as pltpu
from jax.experimental.pallas import tpu_sc as plsc

D_MODEL = 64
NUM_CORES = 2
NUM_SUBCORES = 16
NUM_WORKERS = NUM_CORES * NUM_SUBCORES


def _embed_call(n_rows, chunk, nbuf):
    assert n_rows % NUM_WORKERS == 0
    b_per_w = n_rows // NUM_WORKERS
    assert b_per_w % chunk == 0
    n_chunks = b_per_w // chunk
    assert n_chunks >= nbuf
    width = 2 * D_MODEL

    mesh = plsc.VectorSubcoreMesh(core_axis_name="c", subcore_axis_name="s")

    @functools.partial(
        pl.kernel,
        mesh=mesh,
        out_type=jax.ShapeDtypeStruct((n_rows, width), jnp.float32),
        scratch_types=[
            pltpu.VMEM((b_per_w,), jnp.int32),
            pltpu.VMEM((nbuf, chunk, width), jnp.float32),
            pltpu.SemaphoreType.DMA((nbuf,)),
            pltpu.SemaphoreType.DMA((nbuf,)),
        ],
    )
    def k(idx_hbm, table_hbm, out_hbm, idx_v, rows_v, gsem, ssem):
        wid = lax.axis_index("s") * NUM_CORES + lax.axis_index("c")
        base = wid * b_per_w
        pltpu.sync_copy(idx_hbm.at[pl.ds(base, b_per_w)], idx_v)

        def gather(c):
            b = c % nbuf
            return pltpu.make_async_copy(
                table_hbm.at[idx_v.at[pl.ds(c * chunk, chunk)]],
                rows_v.at[b],
                gsem.at[b],
            )

        def store(c):
            b = c % nbuf
            return pltpu.make_async_copy(
                rows_v.at[b],
                out_hbm.at[pl.ds(base + c * chunk, chunk)],
                ssem.at[b],
            )

        for c in range(nbuf - 1):
            gather(c).start()
        for c in range(n_chunks):
            nxt = c + nbuf - 1
            if nxt < n_chunks:
                if nxt >= nbuf:
                    store(nxt - nbuf).wait()
                gather(nxt).start()
            gather(c).wait()
            store(c).start()
        for c in range(n_chunks - nbuf, n_chunks):
            store(c).wait()

    return k


def _transpose_pad(table_t):
    d, v = table_t.shape
    blk = 4096
    grid = (v + blk - 1) // blk

    def body(t_ref, w_ref):
        t = t_ref[...].T
        w_ref[...] = jnp.concatenate([t, jnp.zeros_like(t)], axis=1)

    return pl.pallas_call(
        body,
        grid=(grid,),
        in_specs=[pl.BlockSpec((d, blk), lambda j: (0, j))],
        out_specs=pl.BlockSpec((blk, 2 * d), lambda j: (j, 0)),
        out_shape=jax.ShapeDtypeStruct((v, 2 * d), jnp.float32),
        compiler_params=pltpu.CompilerParams(
            dimension_semantics=("arbitrary",),
        ),
    )(table_t)


def kernel(x, table):
    b, s = x.shape
    idx = x.T.reshape(-1).astype(jnp.int32)
    table2 = _transpose_pad(table.T)
    wide = _embed_call(idx.shape[0], 200, 4)(idx, table2)
    out = wide[:, :D_MODEL]
    return out.reshape(s, b, D_MODEL).transpose(1, 0, 2)


# speedup vs baseline: 1.5622x; 1.1870x over previous
import functools

import jax
import jax.numpy as jnp
from jax import lax
from jax.experimental import pallas as pl
from jax.experimental.pallas import tpu as pltpu
from jax.experimental.pallas import tpu_sc as plsc

D_MODEL = 64
NUM_CORES = 2
NUM_SUBCORES = 16
NUM_WORKERS = NUM_CORES * NUM_SUBCORES


def _embed_call(n_rows, chunk, nbuf):
    assert n_rows % NUM_WORKERS == 0
    b_per_w = n_rows // NUM_WORKERS
    assert b_per_w % chunk == 0
    n_chunks = b_per_w // chunk
    assert n_chunks >= nbuf
    width = 2 * D_MODEL

    mesh = plsc.VectorSubcoreMesh(core_axis_name="c", subcore_axis_name="s")

    @functools.partial(
        pl.kernel,
        mesh=mesh,
        out_type=jax.ShapeDtypeStruct((n_rows, width), jnp.float32),
        scratch_types=[
            pltpu.VMEM((b_per_w,), jnp.int32),
            pltpu.VMEM((nbuf, chunk, width), jnp.float32),
            pltpu.SemaphoreType.DMA((nbuf,)),
            pltpu.SemaphoreType.DMA((nbuf,)),
        ],
    )
    def k(idx_hbm, table_hbm, out_hbm, idx_v, rows_v, gsem, ssem):
        wid = lax.axis_index("s") * NUM_CORES + lax.axis_index("c")
        base = wid * b_per_w
        pltpu.sync_copy(idx_hbm.at[pl.ds(base, b_per_w)], idx_v)

        def gather(c):
            b = c % nbuf
            return pltpu.make_async_copy(
                table_hbm.at[idx_v.at[pl.ds(c * chunk, chunk)]],
                rows_v.at[b],
                gsem.at[b],
            )

        def store(c):
            b = c % nbuf
            return pltpu.make_async_copy(
                rows_v.at[b],
                out_hbm.at[pl.ds(base + c * chunk, chunk)],
                ssem.at[b],
            )

        for c in range(nbuf - 1):
            gather(c).start()
        for c in range(n_chunks):
            nxt = c + nbuf - 1
            if nxt < n_chunks:
                if nxt >= nbuf:
                    store(nxt - nbuf).wait()
                gather(nxt).start()
            gather(c).wait()
            store(c).start()
        for c in range(n_chunks - nbuf, n_chunks):
            store(c).wait()

    return k


def _transpose_pad(table_t):
    d, v = table_t.shape
    blk = 8192
    grid = (v + blk - 1) // blk

    def body(t_ref, w_ref):
        t = t_ref[...].T
        w_ref[...] = jnp.concatenate([t, jnp.zeros_like(t)], axis=1)

    return pl.pallas_call(
        body,
        grid=(grid,),
        in_specs=[pl.BlockSpec((d, blk), lambda j: (0, j))],
        out_specs=pl.BlockSpec((blk, 2 * d), lambda j: (j, 0)),
        out_shape=jax.ShapeDtypeStruct((v, 2 * d), jnp.float32),
        compiler_params=pltpu.CompilerParams(
            dimension_semantics=("arbitrary",),
        ),
    )(table_t)


def kernel(x, table):
    b, s = x.shape
    idx = x.T.reshape(-1).astype(jnp.int32)
    table2 = _transpose_pad(table.T)
    wide = _embed_call(idx.shape[0], 200, 4)(idx, table2)
    out = wide[:, :D_MODEL]
    return out.reshape(s, b, D_MODEL).transpose(1, 0, 2)


# speedup vs baseline: 1.6417x; 1.0508x over previous
import functools

import jax
import jax.numpy as jnp
from jax import lax
from jax.experimental import pallas as pl
from jax.experimental.pallas import tpu as pltpu
from jax.experimental.pallas import tpu_sc as plsc

D_MODEL = 64
NUM_CORES = 2
NUM_SUBCORES = 16
NUM_WORKERS = NUM_CORES * NUM_SUBCORES


def _embed_call(n_rows, chunk, nbuf):
    assert n_rows % NUM_WORKERS == 0
    b_per_w = n_rows // NUM_WORKERS
    assert b_per_w % chunk == 0
    n_chunks = b_per_w // chunk
    assert n_chunks >= nbuf
    width = 2 * D_MODEL

    mesh = plsc.VectorSubcoreMesh(core_axis_name="c", subcore_axis_name="s")

    @functools.partial(
        pl.kernel,
        mesh=mesh,
        out_type=jax.ShapeDtypeStruct((n_rows, width), jnp.float32),
        scratch_types=[
            pltpu.VMEM((b_per_w,), jnp.int32),
            pltpu.VMEM((nbuf, chunk, width), jnp.float32),
            pltpu.SemaphoreType.DMA((nbuf,)),
            pltpu.SemaphoreType.DMA((nbuf,)),
        ],
    )
    def k(idx_hbm, table_hbm, out_hbm, idx_v, rows_v, gsem, ssem):
        wid = lax.axis_index("s") * NUM_CORES + lax.axis_index("c")
        base = wid * b_per_w
        pltpu.sync_copy(idx_hbm.at[pl.ds(base, b_per_w)], idx_v)

        def gather(c):
            b = c % nbuf
            return pltpu.make_async_copy(
                table_hbm.at[idx_v.at[pl.ds(c * chunk, chunk)]],
                rows_v.at[b],
                gsem.at[b],
            )

        def store(c):
            b = c % nbuf
            return pltpu.make_async_copy(
                rows_v.at[b],
                out_hbm.at[pl.ds(base + c * chunk, chunk)],
                ssem.at[b],
            )

        for c in range(nbuf - 1):
            gather(c).start()
        for c in range(n_chunks):
            nxt = c + nbuf - 1
            if nxt < n_chunks:
                if nxt >= nbuf:
                    store(nxt - nbuf).wait()
                gather(nxt).start()
            gather(c).wait()
            store(c).start()
        for c in range(n_chunks - nbuf, n_chunks):
            store(c).wait()

    return k


def _transpose_pad(table_t):
    d, v = table_t.shape
    blk = 16384
    grid = (v + blk - 1) // blk

    def body(t_ref, w_ref):
        t = t_ref[...].T
        w_ref[...] = jnp.concatenate([t, jnp.zeros_like(t)], axis=1)

    return pl.pallas_call(
        body,
        grid=(grid,),
        in_specs=[pl.BlockSpec((d, blk), lambda j: (0, j))],
        out_specs=pl.BlockSpec((blk, 2 * d), lambda j: (j, 0)),
        out_shape=jax.ShapeDtypeStruct((v, 2 * d), jnp.float32),
        compiler_params=pltpu.CompilerParams(
            dimension_semantics=("arbitrary",),
        ),
    )(table_t)


def kernel(x, table):
    b, s = x.shape
    idx = x.T.reshape(-1).astype(jnp.int32)
    table2 = _transpose_pad(table.T)
    wide = _embed_call(idx.shape[0], 200, 4)(idx, table2)
    out = wide[:, :D_MODEL]
    return out.reshape(s, b, D_MODEL).transpose(1, 0, 2)


# speedup vs baseline: 1.6710x; 1.0179x over previous
import functools

import jax
import jax.numpy as jnp
from jax import lax
from jax.experimental import pallas as pl
from jax.experimental.pallas import tpu as pltpu
from jax.experimental.pallas import tpu_sc as plsc

D_MODEL = 64
NUM_CORES = 2
NUM_SUBCORES = 16
NUM_WORKERS = NUM_CORES * NUM_SUBCORES


def _embed_call(n_rows, chunk, nbuf):
    assert n_rows % NUM_WORKERS == 0
    b_per_w = n_rows // NUM_WORKERS
    assert b_per_w % chunk == 0
    n_chunks = b_per_w // chunk
    assert n_chunks >= nbuf
    width = 2 * D_MODEL

    mesh = plsc.VectorSubcoreMesh(core_axis_name="c", subcore_axis_name="s")

    @functools.partial(
        pl.kernel,
        mesh=mesh,
        out_type=jax.ShapeDtypeStruct((n_rows, width), jnp.float32),
        scratch_types=[
            pltpu.VMEM((b_per_w,), jnp.int32),
            pltpu.VMEM((nbuf, chunk, width), jnp.float32),
            pltpu.SemaphoreType.DMA((nbuf,)),
            pltpu.SemaphoreType.DMA((nbuf,)),
        ],
    )
    def k(idx_hbm, table_hbm, out_hbm, idx_v, rows_v, gsem, ssem):
        wid = lax.axis_index("s") * NUM_CORES + lax.axis_index("c")
        base = wid * b_per_w
        pltpu.sync_copy(idx_hbm.at[pl.ds(base, b_per_w)], idx_v)

        def gather(c):
            b = c % nbuf
            return pltpu.make_async_copy(
                table_hbm.at[idx_v.at[pl.ds(c * chunk, chunk)]],
                rows_v.at[b],
                gsem.at[b],
            )

        def store(c):
            b = c % nbuf
            return pltpu.make_async_copy(
                rows_v.at[b],
                out_hbm.at[pl.ds(base + c * chunk, chunk)],
                ssem.at[b],
            )

        for c in range(nbuf - 1):
            gather(c).start()
        for c in range(n_chunks):
            nxt = c + nbuf - 1
            if nxt < n_chunks:
                if nxt >= nbuf:
                    store(nxt - nbuf).wait()
                gather(nxt).start()
            gather(c).wait()
            store(c).start()
        for c in range(n_chunks - nbuf, n_chunks):
            store(c).wait()

    return k


def _transpose_pad(table_t):
    d, v = table_t.shape
    blk = 32768
    grid = (v + blk - 1) // blk

    def body(t_ref, w_ref):
        t = t_ref[...].T
        w_ref[...] = jnp.concatenate([t, jnp.zeros_like(t)], axis=1)

    return pl.pallas_call(
        body,
        grid=(grid,),
        in_specs=[pl.BlockSpec((d, blk), lambda j: (0, j))],
        out_specs=pl.BlockSpec((blk, 2 * d), lambda j: (j, 0)),
        out_shape=jax.ShapeDtypeStruct((v, 2 * d), jnp.float32),
        compiler_params=pltpu.CompilerParams(
            dimension_semantics=("arbitrary",),
        ),
    )(table_t)


def kernel(x, table):
    b, s = x.shape
    idx = x.T.reshape(-1).astype(jnp.int32)
    table2 = _transpose_pad(table.T)
    wide = _embed_call(idx.shape[0], 200, 4)(idx, table2)
    out = wide[:, :D_MODEL]
    return out.reshape(s, b, D_MODEL).transpose(1, 0, 2)
